# Initial kernel scaffold; baseline (speedup 1.0000x reference)
import jax
import jax.numpy as jnp
from jax.experimental import pallas as pl


def kernel(x, edge_index, W, b, Wc, bc):
    raise NotImplementedError("write your pallas kernel here")


# speedup vs baseline: 113.2650x; 113.2650x over previous
import functools

import jax
import jax.numpy as jnp
from jax import lax
from jax.experimental import pallas as pl
from jax.experimental.pallas import tpu as pltpu
from jax.experimental.pallas import tpu_sc as plsc

N_NODES = 10000
N_EDGES = 320000
NPAD = 10240
NT = 16
EPT = N_EDGES // NT
STRIPE = NPAD // NT
L = 16


def _sc_body(ei_hbm, c_hbm, src_v, dst_v, hist_v, isd_v, acc_v, sacc_v,
             tmp_v, stage_sh, isd_sh):
    w = lax.axis_index("s")
    base = w * EPT
    col = w * STRIPE

    pltpu.sync_copy(ei_hbm.at[pl.ds(base, EPT)], src_v)
    pltpu.sync_copy(ei_hbm.at[pl.ds(N_EDGES + base, EPT)], dst_v)

    zeros16 = jnp.zeros((L,), jnp.float32)
    ones16 = jnp.ones((L,), jnp.float32)

    def zero_hist(i, _):
        hist_v[pl.ds(i * L, L)] = zeros16
        return 0

    lax.fori_loop(0, NPAD // L, zero_hist, 0)

    def phase_a(i, _):
        d16 = dst_v[pl.ds(i * L, L)]
        plsc.addupdate_scatter(hist_v, [d16], ones16)
        return 0

    lax.fori_loop(0, EPT // L, phase_a, 0)

    pltpu.sync_copy(hist_v, stage_sh.at[w])
    plsc.subcore_barrier()

    def zero_acc(j, _):
        acc_v[pl.ds(j * L, L)] = zeros16
        return 0

    lax.fori_loop(0, STRIPE // L, zero_acc, 0)

    def reduce_row(r, _):
        pltpu.sync_copy(stage_sh.at[r, pl.ds(col, STRIPE)], tmp_v)

        def add16(j, _):
            s = pl.ds(j * L, L)
            acc_v[s] = acc_v[s] + tmp_v[s]
            return 0

        lax.fori_loop(0, STRIPE // L, add16, 0)
        return 0

    lax.fori_loop(0, NT, reduce_row, 0)

    def rsqrt16(j, _):
        s = pl.ds(j * L, L)
        xv = acc_v[s] + 1.0
        xi = plsc.bitcast(xv, jnp.int32)
        yi = jnp.int32(0x5F3759DF) - (xi >> 1)
        y = plsc.bitcast(yi, jnp.float32)
        y = y * (1.5 - 0.5 * xv * y * y)
        y = y * (1.5 - 0.5 * xv * y * y)
        y = y * (1.5 - 0.5 * xv * y * y)
        acc_v[s] = y
        return 0

    lax.fori_loop(0, STRIPE // L, rsqrt16, 0)

    pltpu.sync_copy(acc_v, isd_sh.at[pl.ds(col, STRIPE)])
    plsc.subcore_barrier()
    pltpu.sync_copy(isd_sh, isd_v)

    lax.fori_loop(0, NPAD // L, zero_hist, 0)

    def phase_b(i, _):
        sl = pl.ds(i * L, L)
        d16 = dst_v[sl]
        s16 = src_v[sl]
        vals = plsc.load_gather(isd_v, [d16])
        plsc.addupdate_scatter(hist_v, [s16], vals)
        return 0

    lax.fori_loop(0, EPT // L, phase_b, 0)

    pltpu.sync_copy(hist_v, stage_sh.at[w])
    plsc.subcore_barrier()

    def zero_sacc(j, _):
        sacc_v[pl.ds(j * L, L)] = zeros16
        return 0

    lax.fori_loop(0, STRIPE // L, zero_sacc, 0)

    def reduce_row_s(r, _):
        pltpu.sync_copy(stage_sh.at[r, pl.ds(col, STRIPE)], tmp_v)

        def add16(j, _):
            s = pl.ds(j * L, L)
            sacc_v[s] = sacc_v[s] + tmp_v[s]
            return 0

        lax.fori_loop(0, STRIPE // L, add16, 0)
        return 0

    lax.fori_loop(0, NT, reduce_row_s, 0)

    def make_c(j, _):
        s = pl.ds(j * L, L)
        sacc_v[s] = acc_v[s] * (sacc_v[s] + acc_v[s])
        return 0

    lax.fori_loop(0, STRIPE // L, make_c, 0)

    pltpu.sync_copy(sacc_v, c_hbm.at[pl.ds(col, STRIPE)])


_sc_kernel = functools.partial(
    pl.kernel,
    out_type=jax.ShapeDtypeStruct((NPAD,), jnp.float32),
    mesh=plsc.VectorSubcoreMesh(
        core_axis_name="c", subcore_axis_name="s", num_cores=1),
    compiler_params=pltpu.CompilerParams(needs_layout_passes=False),
    scratch_types=[
        pltpu.VMEM((EPT,), jnp.int32),
        pltpu.VMEM((EPT,), jnp.int32),
        pltpu.VMEM((NPAD,), jnp.float32),
        pltpu.VMEM((NPAD,), jnp.float32),
        pltpu.VMEM((STRIPE,), jnp.float32),
        pltpu.VMEM((STRIPE,), jnp.float32),
        pltpu.VMEM((STRIPE,), jnp.float32),
        pltpu.VMEM_SHARED((NT, NPAD), jnp.float32),
        pltpu.VMEM_SHARED((NPAD,), jnp.float32),
    ],
)(_sc_body)


def _dense_body(c_ref, x_ref, w_ref, b_ref, wc_ref, bc_ref, o_ref):
    c = c_ref[...]
    v = jnp.dot(c, x_ref[...], preferred_element_type=jnp.float32)
    sumc = jnp.sum(c)
    pooled = (jnp.dot(v, w_ref[...], preferred_element_type=jnp.float32)
              + sumc * b_ref[...]) * (1.0 / N_NODES)
    o_ref[...] = (jnp.dot(pooled, wc_ref[...],
                          preferred_element_type=jnp.float32) + bc_ref[...])


def kernel(x, edge_index, W, b, Wc, bc):
    ei = edge_index.astype(jnp.int32).reshape(-1)
    c_pad = _sc_kernel(ei)
    c2 = c_pad[:N_NODES].reshape(1, N_NODES)
    logits = pl.pallas_call(
        _dense_body,
        out_shape=jax.ShapeDtypeStruct((1, 10), jnp.float32),
    )(c2, x, W, b.reshape(1, -1), Wc, bc.reshape(1, -1))
    return logits.reshape(10)


# speedup vs baseline: 191.3671x; 1.6896x over previous
import functools

import jax
import jax.numpy as jnp
from jax import lax
from jax.experimental import pallas as pl
from jax.experimental.pallas import tpu as pltpu
from jax.experimental.pallas import tpu_sc as plsc

N_NODES = 10000
N_EDGES = 320000
NPAD = 10240
NT = 16
EPT = N_EDGES // NT
STRIPE = NPAD // NT
L = 16


def _rsqrt16(xv):
    xi = plsc.bitcast(xv, jnp.int32)
    yi = jnp.int32(0x5F3759DF) - (xi >> 1)
    y = plsc.bitcast(yi, jnp.float32)
    y = y * (1.5 - 0.5 * xv * y * y)
    y = y * (1.5 - 0.5 * xv * y * y)
    y = y * (1.5 - 0.5 * xv * y * y)
    return y


def _sc_body(ei_hbm, c_hbm, src_v, dst_v, hist_v, isd_v, acc_v, cbuf_v,
             block_v, stage_sh, isd_sh):
    w = lax.axis_index("s")
    base = w * EPT
    col = w * STRIPE

    pltpu.sync_copy(ei_hbm.at[pl.ds(base, EPT)], src_v)
    pltpu.sync_copy(ei_hbm.at[pl.ds(N_EDGES + base, EPT)], dst_v)

    zeros16 = jnp.zeros((L,), jnp.float32)
    ones16 = jnp.ones((L,), jnp.float32)

    @plsc.parallel_loop(0, NPAD, step=L, unroll=8)
    def _zero_a(i):
        hist_v[pl.ds(i, L)] = zeros16

    @plsc.parallel_loop(0, EPT, step=L, unroll=8)
    def _phase_a(i):
        d16 = dst_v[pl.ds(i, L)]
        plsc.addupdate_scatter(hist_v, [d16], ones16)

    pltpu.sync_copy(hist_v, stage_sh.at[w])
    plsc.subcore_barrier()

    pltpu.sync_copy(stage_sh.at[:, pl.ds(col, STRIPE)], block_v)

    @plsc.parallel_loop(0, STRIPE, step=L, unroll=2)
    def _make_isd(j):
        s = pl.ds(j, L)
        acc = block_v[0, s]
        for r in range(1, NT):
            acc = acc + block_v[r, s]
        acc_v[s] = _rsqrt16(acc + 1.0)

    pltpu.sync_copy(acc_v, isd_sh.at[pl.ds(col, STRIPE)])
    plsc.subcore_barrier()
    pltpu.sync_copy(isd_sh, isd_v)

    @plsc.parallel_loop(0, NPAD, step=L, unroll=8)
    def _zero_b(i):
        hist_v[pl.ds(i, L)] = zeros16

    @plsc.parallel_loop(0, EPT, step=L, unroll=8)
    def _phase_b(i):
        sl = pl.ds(i, L)
        d16 = dst_v[sl]
        s16 = src_v[sl]
        vals = plsc.load_gather(isd_v, [d16])
        plsc.addupdate_scatter(hist_v, [s16], vals)

    pltpu.sync_copy(hist_v, stage_sh.at[w])
    plsc.subcore_barrier()

    pltpu.sync_copy(stage_sh.at[:, pl.ds(col, STRIPE)], block_v)

    @plsc.parallel_loop(0, STRIPE, step=L, unroll=2)
    def _make_c(j):
        s = pl.ds(j, L)
        acc = block_v[0, s]
        for r in range(1, NT):
            acc = acc + block_v[r, s]
        isd16 = acc_v[s]
        cbuf_v[s] = isd16 * (acc + isd16)

    pltpu.sync_copy(cbuf_v, c_hbm.at[pl.ds(col, STRIPE)])


_sc_kernel = functools.partial(
    pl.kernel,
    out_type=jax.ShapeDtypeStruct((NPAD,), jnp.float32),
    mesh=plsc.VectorSubcoreMesh(
        core_axis_name="c", subcore_axis_name="s", num_cores=1),
    compiler_params=pltpu.CompilerParams(needs_layout_passes=False),
    scratch_types=[
        pltpu.VMEM((EPT,), jnp.int32),
        pltpu.VMEM((EPT,), jnp.int32),
        pltpu.VMEM((NPAD,), jnp.float32),
        pltpu.VMEM((NPAD,), jnp.float32),
        pltpu.VMEM((STRIPE,), jnp.float32),
        pltpu.VMEM((STRIPE,), jnp.float32),
        pltpu.VMEM((NT, STRIPE), jnp.float32),
        pltpu.VMEM_SHARED((NT, NPAD), jnp.float32),
        pltpu.VMEM_SHARED((NPAD,), jnp.float32),
    ],
)(_sc_body)


def _dense_body(c_ref, x_ref, w_ref, b_ref, wc_ref, bc_ref, o_ref):
    c = c_ref[...]
    v = jnp.dot(c, x_ref[...], preferred_element_type=jnp.float32)
    sumc = jnp.sum(c)
    pooled = (jnp.dot(v, w_ref[...], preferred_element_type=jnp.float32)
              + sumc * b_ref[...]) * (1.0 / N_NODES)
    o_ref[...] = (jnp.dot(pooled, wc_ref[...],
                          preferred_element_type=jnp.float32) + bc_ref[...])


def kernel(x, edge_index, W, b, Wc, bc):
    ei = edge_index.astype(jnp.int32).reshape(-1)
    c_pad = _sc_kernel(ei)
    c2 = c_pad[:N_NODES].reshape(1, N_NODES)
    logits = pl.pallas_call(
        _dense_body,
        out_shape=jax.ShapeDtypeStruct((1, 10), jnp.float32),
    )(c2, x, W, b.reshape(1, -1), Wc, bc.reshape(1, -1))
    return logits.reshape(10)


# speedup vs baseline: 211.5881x; 1.1057x over previous
import functools

import jax
import jax.numpy as jnp
from jax import lax
from jax.experimental import pallas as pl
from jax.experimental.pallas import tpu as pltpu
from jax.experimental.pallas import tpu_sc as plsc

N_NODES = 10000
N_EDGES = 320000
NPAD = 10240
NT = 16
EPT = 19968
TAIL = N_EDGES - NT * EPT
EBUF = EPT + TAIL
STRIPE = NPAD // NT
L = 16


def _rsqrt16(xv):
    xi = plsc.bitcast(xv, jnp.int32)
    yi = jnp.int32(0x5F3759DF) - (xi >> 1)
    y = plsc.bitcast(yi, jnp.float32)
    y = y * (1.5 - 0.5 * xv * y * y)
    y = y * (1.5 - 0.5 * xv * y * y)
    y = y * (1.5 - 0.5 * xv * y * y)
    return y


def _sc_body(ei_hbm, c_hbm, ei_v, hist_v, isd_v, acc_v, cbuf_v,
             block_v, sem_e, stage_sh, isd_sh):
    w = lax.axis_index("s")
    base = w * EPT
    col = w * STRIPE

    cp_e = pltpu.async_copy(
        ei_hbm.at[:, pl.ds(pl.multiple_of(base, 128), EPT)],
        ei_v.at[:, pl.ds(0, EPT)], sem_e)

    zeros16 = jnp.zeros((L,), jnp.float32)
    ones16 = jnp.ones((L,), jnp.float32)

    @plsc.parallel_loop(0, NPAD, step=L, unroll=8)
    def _zero_a(i):
        hist_v[pl.ds(i, L)] = zeros16

    cp_e.wait()

    @pl.when(w == NT - 1)
    def _tail_copy():
        pltpu.sync_copy(ei_hbm.at[:, pl.ds(NT * EPT, TAIL)],
                        ei_v.at[:, pl.ds(EPT, TAIL)])

    @plsc.parallel_loop(0, EPT, step=L, unroll=8)
    def _phase_a(i):
        d16 = ei_v[1, pl.ds(i, L)]
        plsc.addupdate_scatter(hist_v, [d16], ones16)

    @pl.when(w == NT - 1)
    def _phase_a_tail():
        @plsc.parallel_loop(EPT, EBUF, step=L, unroll=8)
        def _(i):
            d16 = ei_v[1, pl.ds(i, L)]
            plsc.addupdate_scatter(hist_v, [d16], ones16)

    pltpu.sync_copy(hist_v, stage_sh.at[w])
    plsc.subcore_barrier()

    pltpu.sync_copy(stage_sh.at[:, pl.ds(col, STRIPE)], block_v)

    @plsc.parallel_loop(0, STRIPE, step=L, unroll=2)
    def _make_isd(j):
        s = pl.ds(j, L)
        acc = block_v[0, s]
        for r in range(1, NT):
            acc = acc + block_v[r, s]
        acc_v[s] = _rsqrt16(acc + 1.0)

    pltpu.sync_copy(acc_v, isd_sh.at[pl.ds(col, STRIPE)])
    plsc.subcore_barrier()
    pltpu.sync_copy(isd_sh, isd_v)

    @plsc.parallel_loop(0, NPAD, step=L, unroll=8)
    def _zero_b(i):
        hist_v[pl.ds(i, L)] = zeros16

    @plsc.parallel_loop(0, EPT, step=L, unroll=8)
    def _phase_b(i):
        sl = pl.ds(i, L)
        d16 = ei_v[1, sl]
        s16 = ei_v[0, sl]
        vals = plsc.load_gather(isd_v, [d16])
        plsc.addupdate_scatter(hist_v, [s16], vals)

    @pl.when(w == NT - 1)
    def _phase_b_tail():
        @plsc.parallel_loop(EPT, EBUF, step=L, unroll=8)
        def _(i):
            sl = pl.ds(i, L)
            d16 = ei_v[1, sl]
            s16 = ei_v[0, sl]
            vals = plsc.load_gather(isd_v, [d16])
            plsc.addupdate_scatter(hist_v, [s16], vals)

    pltpu.sync_copy(hist_v, stage_sh.at[w])
    plsc.subcore_barrier()

    pltpu.sync_copy(stage_sh.at[:, pl.ds(col, STRIPE)], block_v)

    @plsc.parallel_loop(0, STRIPE, step=L, unroll=2)
    def _make_c(j):
        s = pl.ds(j, L)
        acc = block_v[0, s]
        for r in range(1, NT):
            acc = acc + block_v[r, s]
        isd16 = acc_v[s]
        cbuf_v[s] = isd16 * (acc + isd16)

    pltpu.sync_copy(cbuf_v, c_hbm.at[0, pl.ds(col, STRIPE)])


_sc_kernel = functools.partial(
    pl.kernel,
    out_type=jax.ShapeDtypeStruct((1, NPAD), jnp.float32),
    mesh=plsc.VectorSubcoreMesh(
        core_axis_name="c", subcore_axis_name="s", num_cores=1),
    compiler_params=pltpu.CompilerParams(needs_layout_passes=False),
    scratch_types=[
        pltpu.VMEM((2, EBUF), jnp.int32),
        pltpu.VMEM((NPAD,), jnp.float32),
        pltpu.VMEM((NPAD,), jnp.float32),
        pltpu.VMEM((STRIPE,), jnp.float32),
        pltpu.VMEM((STRIPE,), jnp.float32),
        pltpu.VMEM((NT, STRIPE), jnp.float32),
        pltpu.SemaphoreType.DMA,
        pltpu.VMEM_SHARED((NT, NPAD), jnp.float32),
        pltpu.VMEM_SHARED((NPAD,), jnp.float32),
    ],
)(_sc_body)


def _dense_body(c_ref, x_ref, w_ref, b_ref, wc_ref, bc_ref, o_ref):
    c = c_ref[...][:, :N_NODES]
    v = jnp.dot(c, x_ref[...], preferred_element_type=jnp.float32)
    sumc = jnp.sum(c)
    pooled = (jnp.dot(v, w_ref[...], preferred_element_type=jnp.float32)
              + sumc * b_ref[...]) * (1.0 / N_NODES)
    o_ref[...] = (jnp.dot(pooled, wc_ref[...],
                          preferred_element_type=jnp.float32) + bc_ref[...])


def kernel(x, edge_index, W, b, Wc, bc):
    ei = edge_index.astype(jnp.int32)
    c_pad = _sc_kernel(ei)
    logits = pl.pallas_call(
        _dense_body,
        out_shape=jax.ShapeDtypeStruct((1, 10), jnp.float32),
    )(c_pad, x, W, b.reshape(1, -1), Wc, bc.reshape(1, -1))
    return logits.reshape(10)
